# baseline (speedup 1.0000x reference)
import dataclasses

import jax
import jax.numpy as jnp
from jax import lax
from jax.experimental import pallas as pl
from jax.experimental.pallas import tpu as pltpu
from jax.experimental.pallas import tpu_sc as plsc

_CP = pltpu.CompilerParams()
if "needs_layout_passes" in pltpu.CompilerParams.__dataclass_fields__:
    _CP = dataclasses.replace(_CP, needs_layout_passes=False)

_B = 16
_MAX_LEN = 4096
_D = 512
_TOTAL = 32768

_P = 32
_NW = 32
_NPIECES = (_B * _MAX_LEN) // _P
_PER_W = _NPIECES // _NW
_NSLOT = 6
_LOOKAHEAD = 3

_ZB = 256


def _pad_and_stack_sc(flat, cu16):
    mesh = plsc.VectorSubcoreMesh(core_axis_name="c", subcore_axis_name="s")

    @pl.kernel(
        out_type=jax.ShapeDtypeStruct((_B * _MAX_LEN, _D), jnp.float32),
        mesh=mesh,
        compiler_params=_CP,
        scratch_types=(
            [pltpu.VMEM((_P, _D), jnp.float32)] * _NSLOT
            + [pltpu.VMEM((_NSLOT, _P), jnp.int32)]
            + [pltpu.VMEM((16,), jnp.int32)]
            + [pltpu.SMEM((17,), jnp.int32)]
            + [pltpu.SemaphoreType.DMA] * _NSLOT
            + [pltpu.SemaphoreType.DMA] * _NSLOT
        ),
    )
    def k(flat_hbm, cu_hbm, out_hbm, *scratch):
        bufs = scratch[0:_NSLOT]
        idx_v = scratch[_NSLOT]
        cu_v = scratch[_NSLOT + 1]
        cu_s = scratch[_NSLOT + 2]
        in_sems = scratch[_NSLOT + 3:2 * _NSLOT + 3]
        out_sems = scratch[2 * _NSLOT + 3:3 * _NSLOT + 3]

        wid = lax.axis_index("s") * 2 + lax.axis_index("c")

        pltpu.sync_copy(cu_hbm.at[pl.ds(0, 16)], cu_v)
        cuvec = cu_v[...]
        lane = lax.iota(jnp.int32, 16)
        for i in range(16):
            cu_s[i] = jnp.sum(jnp.where(lane == i, cuvec, 0))
        cu_s[16] = _TOTAL

        def params(i):
            pidx = i * _NW + wid
            row0 = pidx * _P
            b = row0 // _MAX_LEN
            m0 = row0 % _MAX_LEN
            cu_b = cu_s[b]
            cu_b1 = cu_s[b + 1]
            nv = jnp.clip(cu_b1 - cu_b - m0, 0, _P)
            src = cu_b + m0
            return row0, nv, src

        def issue_read(i, s):
            row0, nv, src = params(i)

            @pl.when(nv > 0)
            def _():
                for q in range(_P // 16):
                    idx_v[s, pl.ds(q * 16, 16)] = jnp.minimum(
                        src + lane + (q * 16), _TOTAL - 1)
                pltpu.make_async_copy(
                    flat_hbm.at[idx_v.at[s]], bufs[s], in_sems[s]).start()

        def finish_piece(i, s):
            row0, nv, src = params(i)

            @pl.when(nv > 0)
            def _data():
                pltpu.make_async_copy(
                    flat_hbm.at[idx_v.at[s]], bufs[s], in_sems[s]).wait()

                @pl.loop(nv, _P)
                def _zero_tail(r):
                    for j in range(_D // 16):
                        bufs[s][r, pl.ds(j * 16, 16)] = jnp.zeros(
                            (16,), jnp.float32)

                pltpu.make_async_copy(
                    bufs[s], out_hbm.at[pl.ds(row0, _P)], out_sems[s]).start()

        def drain_write(j, s):
            _, nv, _ = params(j)

            @pl.when(nv > 0)
            def _():
                pltpu.make_async_copy(
                    bufs[s], out_hbm.at[pl.ds(0, _P)], out_sems[s]).wait()

        def body(i, s, next_slot):
            finish_piece(i, s)
            if next_slot is not None:
                j = jnp.maximum(i - (_NSLOT - _LOOKAHEAD), 0)
                _, nv_j, _ = params(j)

                @pl.when((i >= _NSLOT - _LOOKAHEAD) & (nv_j > 0))
                def _():
                    pltpu.make_async_copy(
                        bufs[next_slot], out_hbm.at[pl.ds(0, _P)],
                        out_sems[next_slot]).wait()

                issue_read(i + _LOOKAHEAD, next_slot)

        for p in range(_LOOKAHEAD):
            issue_read(p, p)

        _STEADY_END = ((_PER_W - _LOOKAHEAD) // _NSLOT) * _NSLOT

        @pl.loop(0, _STEADY_END, step=_NSLOT)
        def _steady(i):
            for d in range(_NSLOT):
                body(i + d, d, (d + _LOOKAHEAD) % _NSLOT)

        for p in range(_STEADY_END, _PER_W):
            s = p % _NSLOT
            nxt = p + _LOOKAHEAD
            body(p, s, nxt % _NSLOT if nxt < _PER_W else None)

        for p in range(_PER_W - _NSLOT, _PER_W):
            drain_write(p, p % _NSLOT)

    return k(flat, cu16)


def _tc_pad_body(cu_ref, data_ref, out_ref, zbuf, sem):
    del data_ref
    zbuf[...] = jnp.zeros_like(zbuf)

    total = jnp.int32(0)
    tails = []
    for b in range(_B):
        seg = cu_ref[b + 1] - cu_ref[b]
        datarows = jnp.clip(seg, 0, _MAX_LEN)
        padstart = (datarows + 31) & ~31
        rem = (_MAX_LEN - padstart) % _ZB
        base = b * _MAX_LEN + padstart
        nfull = (_MAX_LEN - padstart) // _ZB

        def _issue(kk, c, base=base):
            o = pl.multiple_of(base + kk * _ZB, 32)
            pltpu.make_async_copy(
                zbuf, out_ref.at[pl.ds(o, _ZB)], sem).start()
            return c

        lax.fori_loop(0, nfull, _issue, 0)
        total = total + nfull
        tails.append((base + nfull * _ZB, rem))

    def _drain(kk, c):
        pltpu.make_async_copy(zbuf, out_ref.at[pl.ds(0, _ZB)], sem).wait()
        return c

    lax.fori_loop(0, total, _drain, 0)

    for tbase, rem in tails:
        off = jnp.int32(0)
        for sz in (128, 64, 32):
            bit = (rem & sz) != 0
            o = off

            @pl.when(bit)
            def _():
                to = pl.multiple_of(tbase + o, 32)
                cp = pltpu.make_async_copy(
                    zbuf.at[pl.ds(0, sz)],
                    out_ref.at[pl.ds(to, sz)], sem)
                cp.start()
                cp.wait()

            off = off + jnp.where(bit, sz, 0)


def _tc_pad_fill(cu17, data):
    return pl.pallas_call(
        _tc_pad_body,
        out_shape=jax.ShapeDtypeStruct((_B * _MAX_LEN, _D), jnp.float32),
        in_specs=[
            pl.BlockSpec(memory_space=pltpu.SMEM),
            pl.BlockSpec(memory_space=pl.ANY),
        ],
        out_specs=pl.BlockSpec(memory_space=pl.ANY),
        input_output_aliases={1: 0},
        scratch_shapes=[
            pltpu.VMEM((_ZB, _D), jnp.float32),
            pltpu.SemaphoreType.DMA,
        ],
    )(cu17, data)


@jax.jit
def kernel(flat, cu_seqlens):
    cu16 = cu_seqlens[:16]
    data = _pad_and_stack_sc(flat, cu16)
    out = _tc_pad_fill(cu_seqlens, data)
    return out.reshape(_B, _MAX_LEN, _D)


# speedup vs baseline: 1.1556x; 1.1556x over previous
import dataclasses

import jax
import jax.numpy as jnp
from jax import lax
from jax.experimental import pallas as pl
from jax.experimental.pallas import tpu as pltpu
from jax.experimental.pallas import tpu_sc as plsc

_CP = pltpu.CompilerParams()
if "needs_layout_passes" in pltpu.CompilerParams.__dataclass_fields__:
    _CP = dataclasses.replace(_CP, needs_layout_passes=False)

_B = 16
_MAX_LEN = 4096
_D = 512
_TOTAL = 32768

_P = 32
_NW = 32
_NPIECES = (_B * _MAX_LEN) // _P
_PER_W = _NPIECES // _NW
_NSLOT = 6
_LOOKAHEAD = 3


def _pad_and_stack_sc(flat, cu16):
    mesh = plsc.VectorSubcoreMesh(core_axis_name="c", subcore_axis_name="s")

    @pl.kernel(
        out_type=jax.ShapeDtypeStruct((_B * _MAX_LEN, _D), jnp.float32),
        mesh=mesh,
        compiler_params=_CP,
        scratch_types=(
            [pltpu.VMEM((_P, _D), jnp.float32)] * _NSLOT
            + [pltpu.VMEM((_P, _D), jnp.float32)]
            + [pltpu.VMEM((_NSLOT, _P), jnp.int32)]
            + [pltpu.VMEM((16,), jnp.int32)]
            + [pltpu.SMEM((17,), jnp.int32)]
            + [pltpu.SemaphoreType.DMA] * _NSLOT
            + [pltpu.SemaphoreType.DMA] * _NSLOT
        ),
    )
    def k(flat_hbm, cu_hbm, out_hbm, *scratch):
        bufs = scratch[0:_NSLOT]
        zbuf = scratch[_NSLOT]
        idx_v = scratch[_NSLOT + 1]
        cu_v = scratch[_NSLOT + 2]
        cu_s = scratch[_NSLOT + 3]
        in_sems = scratch[_NSLOT + 4:2 * _NSLOT + 4]
        out_sems = scratch[2 * _NSLOT + 4:3 * _NSLOT + 4]

        wid = lax.axis_index("s") * 2 + lax.axis_index("c")

        @pl.loop(0, _P)
        def _zero_row(r):
            for j in range(_D // 16):
                zbuf[r, pl.ds(j * 16, 16)] = jnp.zeros((16,), jnp.float32)

        pltpu.sync_copy(cu_hbm.at[pl.ds(0, 16)], cu_v)
        cuvec = cu_v[...]
        lane = lax.iota(jnp.int32, 16)
        for i in range(16):
            cu_s[i] = jnp.sum(jnp.where(lane == i, cuvec, 0))
        cu_s[16] = _TOTAL

        def params(i):
            pidx = i * _NW + wid
            row0 = pidx * _P
            b = row0 // _MAX_LEN
            m0 = row0 % _MAX_LEN
            cu_b = cu_s[b]
            cu_b1 = cu_s[b + 1]
            nv = jnp.clip(cu_b1 - cu_b - m0, 0, _P)
            src = cu_b + m0
            return row0, nv, src

        def issue_read(i, s):
            row0, nv, src = params(i)

            @pl.when(nv > 0)
            def _():
                for q in range(_P // 16):
                    idx_v[s, pl.ds(q * 16, 16)] = jnp.minimum(
                        src + lane + (q * 16), _TOTAL - 1)
                pltpu.make_async_copy(
                    flat_hbm.at[idx_v.at[s]], bufs[s], in_sems[s]).start()

        def finish_piece(i, s):
            row0, nv, src = params(i)

            @pl.when(nv > 0)
            def _data():
                pltpu.make_async_copy(
                    flat_hbm.at[idx_v.at[s]], bufs[s], in_sems[s]).wait()

                @pl.loop(nv, _P)
                def _zero_tail(r):
                    for j in range(_D // 16):
                        bufs[s][r, pl.ds(j * 16, 16)] = jnp.zeros(
                            (16,), jnp.float32)

                pltpu.make_async_copy(
                    bufs[s], out_hbm.at[pl.ds(row0, _P)], out_sems[s]).start()

            @pl.when(nv == 0)
            def _all_pad():
                pltpu.make_async_copy(
                    zbuf, out_hbm.at[pl.ds(row0, _P)], out_sems[s]).start()

        def drain_write(s):
            pltpu.make_async_copy(
                zbuf, out_hbm.at[pl.ds(0, _P)], out_sems[s]).wait()

        def body(i, s, next_slot):
            finish_piece(i, s)
            if next_slot is not None:
                @pl.when(i >= _NSLOT - _LOOKAHEAD)
                def _():
                    drain_write(next_slot)

                issue_read(i + _LOOKAHEAD, next_slot)

        for p in range(_LOOKAHEAD):
            issue_read(p, p)

        _STEADY_END = ((_PER_W - _LOOKAHEAD) // _NSLOT) * _NSLOT

        @pl.loop(0, _STEADY_END, step=_NSLOT)
        def _steady(i):
            for d in range(_NSLOT):
                body(i + d, d, (d + _LOOKAHEAD) % _NSLOT)

        for p in range(_STEADY_END, _PER_W):
            s = p % _NSLOT
            nxt = p + _LOOKAHEAD
            body(p, s, nxt % _NSLOT if nxt < _PER_W else None)

        for p in range(_PER_W - _NSLOT, _PER_W):
            drain_write(p % _NSLOT)

    return k(flat, cu16)


@jax.jit
def kernel(flat, cu_seqlens):
    cu16 = cu_seqlens[:16]
    out = _pad_and_stack_sc(flat, cu16)
    return out.reshape(_B, _MAX_LEN, _D)
